# baseline (speedup 1.0000x reference)
import functools

import jax
import jax.numpy as jnp
from jax import lax
from jax.experimental import pallas as pl
from jax.experimental.pallas import tpu as pltpu
from jax.experimental.pallas import tpu_sc as plsc

NC = 2
NS = 16
NW = NC * NS
NSEQ = 8


@functools.partial(jax.jit, static_argnames=("d",))
def _gather(table, xi, *, d):
    n_seq, seq_len = xi.shape
    assert seq_len <= 128
    s_per_w = n_seq // NW
    n_groups = s_per_w // NSEQ
    assert n_groups % 2 == 0
    mesh = plsc.VectorSubcoreMesh(core_axis_name="c", subcore_axis_name="s")

    @functools.partial(
        pl.kernel,
        mesh=mesh,
        compiler_params=pltpu.CompilerParams(use_tc_tiling_on_sc=False),
        out_type=jax.ShapeDtypeStruct((n_seq, seq_len, d), jnp.float32),
        scratch_types=[
            pltpu.VMEM((s_per_w, seq_len), jnp.int32),
            pltpu.VMEM((2, NSEQ, seq_len, d), jnp.float32),
            [pltpu.SemaphoreType.DMA] * 2,
            [pltpu.SemaphoreType.DMA] * 2,
        ],
    )
    def k(table_hbm, idx_hbm, out_hbm, idx_v, rows_v, gsems, wsems):
        wid = lax.axis_index("s") * NC + lax.axis_index("c")
        base = wid * s_per_w
        pltpu.sync_copy(idx_hbm.at[pl.ds(base, s_per_w)], idx_v)

        def pair(g2, carry):
            for p in range(2):
                g = 2 * g2 + p

                @pl.when(g2 > 0)
                def _wait_write(p=p):
                    pltpu.make_async_copy(
                        rows_v.at[p], out_hbm.at[pl.ds(0, NSEQ)], wsems[p]
                    ).wait()

                for q in range(NSEQ):
                    pltpu.async_copy(
                        table_hbm.at[idx_v.at[g * NSEQ + q]],
                        rows_v.at[p, q],
                        gsems[p],
                    )
            for p in range(2):
                g = 2 * g2 + p
                pltpu.make_async_copy(
                    out_hbm.at[pl.ds(0, NSEQ)], rows_v.at[p], gsems[p]
                ).wait()
                pltpu.async_copy(
                    rows_v.at[p],
                    out_hbm.at[pl.ds(base + g * NSEQ, NSEQ)],
                    wsems[p],
                )
            return carry

        lax.fori_loop(0, n_groups // 2, pair, 0)
        for p in range(2):
            pltpu.make_async_copy(
                rows_v.at[p], out_hbm.at[pl.ds(0, NSEQ)], wsems[p]
            ).wait()

    return k(table, xi)


def kernel(x, table):
    n_seq, seq_len = x.shape
    d = table.shape[1]
    xi = x.astype(jnp.int32)
    per = NW * 2 * NSEQ
    pad = (-n_seq) % per
    if pad:
        xi = jnp.concatenate([xi, jnp.zeros((pad, seq_len), jnp.int32)])
    out = _gather(table, xi, d=d)
    if pad:
        out = out[:n_seq]
    return out
